# baseline (speedup 1.0000x reference)
import functools

import jax
import jax.numpy as jnp
from jax import lax
from jax.experimental import pallas as pl
from jax.experimental.pallas import tpu as pltpu
from jax.experimental.pallas import tpu_sc as plsc

_K = 10000
_NC, _NS, _L = 2, 16, 16
_NW = _NC * _NS
_N = 128 * 32768
_PER_W = _N // _NW
_CHUNK = 16384
_NCHUNK = _PER_W // _CHUNK

_B1, _B2, _B3 = 10, 11, 11


def _iota16():
    return lax.broadcasted_iota(jnp.int32, (_L,), 0)


def _keys_kernel(x_ref, o_ref):
    b = lax.bitcast_convert_type(x_ref[...], jnp.int32)
    skey = b ^ ((b >> 31) & jnp.int32(0x7FFFFFFF))
    o_ref[...] = skey.reshape(o_ref.shape)


def _keys(x):
    rows, cols = x.shape
    return pl.pallas_call(
        _keys_kernel,
        grid=(rows // 8,),
        in_specs=[pl.BlockSpec((8, cols), lambda r: (r, 0))],
        out_specs=pl.BlockSpec((8 * cols,), lambda r: (r,)),
        out_shape=jax.ShapeDtypeStruct((rows * cols,), jnp.int32),
    )(x)


def _sc_pick(read_vec, nvec, k):
    def body(i, st):
        found, bstar, carry, snext = st
        idx = nvec - 1 - i
        v = read_vec(idx)
        svec = lax.rev(plsc.cumsum(lax.rev(v, (0,))), (0,)) + carry
        mask = svec >= k
        m = jnp.max(jnp.where(mask, _iota16(), -1))
        sn_v = jnp.max(jnp.where(mask, 0, svec))
        sn_v = jnp.where(m == _L - 1, carry, sn_v)
        hit = jnp.logical_and(m >= 0, found == 0)
        bstar = jnp.where(hit, idx * _L + m, bstar)
        snext = jnp.where(hit, sn_v, snext)
        found = jnp.where(m >= 0, 1, found)
        carry = svec[0]
        return found, bstar, carry, snext

    _, bstar, _, snext = lax.fori_loop(
        0, nvec, body, (jnp.int32(0), jnp.int32(0), jnp.int32(0), jnp.int32(0))
    )
    return bstar, k - snext


def _make_sc_pass(prior_bits, shift, bits, emit_keys=False):
    nbins = 1 << bits
    rows = nbins // _L
    nprior = len(prior_bits)
    pshift = shift + bits
    pwidth = 32 - pshift
    binflip = (nbins >> 1) if shift + bits == 32 else 0
    mesh = plsc.VectorSubcoreMesh(core_axis_name="c", subcore_axis_name="s")

    scratch = [
        pltpu.VMEM((_NC, 1 << pb), jnp.int32) for pb in prior_bits
    ]
    scratch += [
        pltpu.VMEM((nbins,), jnp.int32),
        pltpu.VMEM((nbins,), jnp.int32),
        pltpu.VMEM((_CHUNK,), jnp.float32 if emit_keys else jnp.int32),
        pltpu.VMEM((_CHUNK,), jnp.float32 if emit_keys else jnp.int32),
        pltpu.VMEM_SHARED((_NS, nbins), jnp.int32),
        pltpu.SemaphoreType.DMA,
        pltpu.SemaphoreType.DMA,
    ]
    if emit_keys:
        scratch += [
            pltpu.VMEM((_CHUNK,), jnp.int32),
            pltpu.VMEM((_CHUNK,), jnp.int32),
            pltpu.SemaphoreType.DMA,
            pltpu.SemaphoreType.DMA,
        ]
        out_type = (
            jax.ShapeDtypeStruct((_NC, nbins), jnp.int32),
            jax.ShapeDtypeStruct((_N,), jnp.int32),
        )
    else:
        out_type = jax.ShapeDtypeStruct((_NC, nbins), jnp.int32)

    @functools.partial(
        pl.kernel,
        out_type=out_type,
        mesh=mesh,
        scratch_types=scratch,
        compiler_params=pltpu.CompilerParams(needs_layout_passes=False),
    )
    def sc_pass(*refs):
        keys_hbm = refs[0]
        prior_hbm = refs[1 : 1 + nprior]
        out_hbm = refs[1 + nprior]
        nout = 2 if emit_keys else 1
        keys_out = refs[2 + nprior] if emit_keys else None
        base_i = 1 + nprior + nout
        prior_v = refs[base_i : base_i + nprior]
        hist, tmp, buf0, buf1, shared = refs[base_i + nprior : base_i + nprior + 5]
        rest = refs[base_i + nprior + 5 :]
        sem0, sem1 = rest[0], rest[1]
        if emit_keys:
            kbufs, ksems = (rest[2], rest[3]), (rest[4], rest[5])
        bufs, sems = (buf0, buf1), (sem0, sem1)

        c = lax.axis_index("c")
        s = lax.axis_index("s")
        zeros = jnp.zeros((_L,), jnp.int32)
        wid = c * _NS + s
        base = wid * _PER_W
        cpr = 32768 // _CHUNK
        rpw = 128 // _NW

        def mk(j, buf, sem):
            if emit_keys:
                row = wid * rpw + j // cpr
                col = (j % cpr) * _CHUNK
                src = keys_hbm.at[row].at[pl.ds(col, _CHUNK)]
            else:
                src = keys_hbm.at[pl.ds(base + j * _CHUNK, _CHUNK)]
            return pltpu.make_async_copy(src, buf, sem)

        bufs, sems = (buf0, buf1), (sem0, sem1)
        mk(0, bufs[0], sems[0]).start()
        mk(1, bufs[1], sems[1]).start()

        def zbody(i, _):
            hist[pl.ds(i * _L, _L)] = zeros
            return 0

        lax.fori_loop(0, rows, zbody, 0)

        k = jnp.int32(_K)
        pfx = jnp.int32(0)
        for p in range(nprior):
            prows = (1 << prior_bits[p]) // _L
            pltpu.sync_copy(prior_hbm[p], prior_v[p])
            pv = prior_v[p]

            def rd(idx, pv=pv):
                return pv[0, pl.ds(idx * _L, _L)] + pv[1, pl.ds(idx * _L, _L)]

            b, k = _sc_pick(rd, prows, k)
            pfx = (pfx << prior_bits[p]) | b

        if nprior:
            ext = 32 - pwidth
            pfx_se = ((pfx ^ (1 << (pwidth - 1))) << ext) >> ext

        ones = zeros + 1
        kcps = [None, None]
        for j in range(_NCHUNK):
            cur = j % 2
            mk(j, bufs[cur], sems[cur]).wait()
            buf = bufs[cur]

            if emit_keys:
                if kcps[cur] is not None:
                    kcps[cur].wait()
                kbuf = kbufs[cur]

                @plsc.parallel_loop(0, _CHUNK, _L, unroll=4)
                def vbody(v, buf=buf, kbuf=kbuf):
                    b = plsc.bitcast(buf[pl.ds(v, _L)], jnp.int32)
                    skey = b ^ ((b >> 31) & jnp.int32(0x7FFFFFFF))
                    kbuf[pl.ds(v, _L)] = skey
                    bin_i = ((skey >> shift) & (nbins - 1)) ^ binflip
                    plsc.addupdate_scatter(hist, [bin_i], ones)

                kcps[cur] = pltpu.make_async_copy(
                    kbuf, keys_out.at[pl.ds(base + j * _CHUNK, _CHUNK)], ksems[cur]
                )
                kcps[cur].start()
            else:

                @plsc.parallel_loop(0, _CHUNK, _L, unroll=4)
                def vbody(v, buf=buf):
                    skey = buf[pl.ds(v, _L)]
                    bin_i = ((skey >> shift) & (nbins - 1)) ^ binflip
                    if nprior:
                        keep = (skey >> pshift) == pfx_se
                        plsc.addupdate_scatter(hist, [bin_i], ones, mask=keep)
                    else:
                        plsc.addupdate_scatter(hist, [bin_i], ones)

            if j + 2 < _NCHUNK:
                mk(j + 2, bufs[cur], sems[cur]).start()

        if emit_keys:
            for kcp in kcps:
                if kcp is not None:
                    kcp.wait()

        pltpu.sync_copy(hist, shared.at[s])
        plsc.subcore_barrier()
        for step in (8, 4, 2, 1):

            @pl.when(s < step)
            def _(step=step):
                pltpu.sync_copy(shared.at[s + step], tmp)

                @plsc.parallel_loop(0, nbins, _L, unroll=4)
                def abody(i):
                    hist[pl.ds(i, _L)] = hist[pl.ds(i, _L)] + tmp[pl.ds(i, _L)]

                pltpu.sync_copy(hist, shared.at[s])

            plsc.subcore_barrier()

        @pl.when(s == 0)
        def _():
            pltpu.sync_copy(hist, out_hbm.at[c])

    return sc_pass


_pass1 = _make_sc_pass([], 22, _B1, emit_keys=True)
_pass2 = _make_sc_pass([_B1], 11, _B2)
_pass3 = _make_sc_pass([_B1, _B2], 0, _B3)


def _tc_pick(h, k):
    nbins = h.shape[1]
    tot = jnp.sum(h, axis=0, keepdims=True)
    iot = lax.broadcasted_iota(jnp.int32, tot.shape, 1)
    zero = jnp.zeros_like(tot)

    def suffix(b):
        return jnp.sum(jnp.where(iot >= b, tot, zero))

    b = jnp.int32(0)
    bit = nbins >> 1
    while bit:
        cand = b + bit
        b = jnp.where(suffix(cand) >= k, cand, b)
        bit >>= 1
    return b, k - suffix(b + 1)


def _mask_kernel(x_ref, h1_ref, h2_ref, h3_ref, o_ref, tsm):
    @pl.when(pl.program_id(0) == 0)
    def _():
        b1, k1 = _tc_pick(h1_ref[...], jnp.int32(_K))
        b2, k2 = _tc_pick(h2_ref[...], k1)
        b3, _ = _tc_pick(h3_ref[...], k2)
        t = (
            (b1.astype(jnp.uint32) << (_B2 + _B3))
            | (b2.astype(jnp.uint32) << _B3)
            | b3.astype(jnp.uint32)
        )
        bits = jnp.where(
            t >= jnp.uint32(0x80000000), t ^ jnp.uint32(0x80000000), ~t
        )
        tsm[0] = lax.bitcast_convert_type(bits, jnp.float32)

    o_ref[...] = (x_ref[...] >= tsm[0]).astype(jnp.float32)


def kernel(x):
    h1, keys = _pass1(x)
    h2 = _pass2(keys, h1)
    h3 = _pass3(keys, h1, h2)
    rows, cols = x.shape
    blk = 8
    return pl.pallas_call(
        _mask_kernel,
        grid=(rows // blk,),
        in_specs=[
            pl.BlockSpec((blk, cols), lambda r: (r, 0)),
            pl.BlockSpec((_NC, 1 << _B1), lambda r: (0, 0)),
            pl.BlockSpec((_NC, 1 << _B2), lambda r: (0, 0)),
            pl.BlockSpec((_NC, 1 << _B3), lambda r: (0, 0)),
        ],
        out_specs=pl.BlockSpec((blk, cols), lambda r: (r, 0)),
        scratch_shapes=[pltpu.SMEM((1,), jnp.float32)],
        out_shape=jax.ShapeDtypeStruct(x.shape, jnp.float32),
    )(x, h1, h2, h3)


# speedup vs baseline: 1.0355x; 1.0355x over previous
import functools

import jax
import jax.numpy as jnp
from jax import lax
from jax.experimental import pallas as pl
from jax.experimental.pallas import tpu as pltpu
from jax.experimental.pallas import tpu_sc as plsc

_K = 10000
_NC, _NS, _L = 2, 16, 16
_NW = _NC * _NS
_N = 128 * 32768
_PER_W = _N // _NW


_B1, _B2, _B3 = 10, 11, 11


def _iota16():
    return lax.broadcasted_iota(jnp.int32, (_L,), 0)


def _keys_kernel(x_ref, o_ref):
    b = lax.bitcast_convert_type(x_ref[...], jnp.int32)
    skey = b ^ ((b >> 31) & jnp.int32(0x7FFFFFFF))
    o_ref[...] = skey.reshape(o_ref.shape)


def _keys(x):
    rows, cols = x.shape
    return pl.pallas_call(
        _keys_kernel,
        grid=(rows // 8,),
        in_specs=[pl.BlockSpec((8, cols), lambda r: (r, 0))],
        out_specs=pl.BlockSpec((8 * cols,), lambda r: (r,)),
        out_shape=jax.ShapeDtypeStruct((rows * cols,), jnp.int32),
    )(x)


def _sc_pick(read_vec, nvec, k):
    def body(i, st):
        found, bstar, carry, snext = st
        idx = nvec - 1 - i
        v = read_vec(idx)
        svec = lax.rev(plsc.cumsum(lax.rev(v, (0,))), (0,)) + carry
        mask = svec >= k
        m = jnp.max(jnp.where(mask, _iota16(), -1))
        sn_v = jnp.max(jnp.where(mask, 0, svec))
        sn_v = jnp.where(m == _L - 1, carry, sn_v)
        hit = jnp.logical_and(m >= 0, found == 0)
        bstar = jnp.where(hit, idx * _L + m, bstar)
        snext = jnp.where(hit, sn_v, snext)
        found = jnp.where(m >= 0, 1, found)
        carry = svec[0]
        return found, bstar, carry, snext

    _, bstar, _, snext = lax.fori_loop(
        0, nvec, body, (jnp.int32(0), jnp.int32(0), jnp.int32(0), jnp.int32(0))
    )
    return bstar, k - snext


def _make_sc_pass(prior_bits, shift, bits, emit_keys=False):
    nbins = 1 << bits
    rows = nbins // _L
    nprior = len(prior_bits)
    _CHUNK = 16384 if emit_keys else 32768
    _NCHUNK = _PER_W // _CHUNK
    pshift = shift + bits
    pwidth = 32 - pshift
    binflip = (nbins >> 1) if shift + bits == 32 else 0
    mesh = plsc.VectorSubcoreMesh(core_axis_name="c", subcore_axis_name="s")

    scratch = [
        pltpu.VMEM((_NC, 1 << pb), jnp.int32) for pb in prior_bits
    ]
    scratch += [
        pltpu.VMEM((nbins,), jnp.int32),
        pltpu.VMEM((nbins,), jnp.int32),
        pltpu.VMEM((_CHUNK,), jnp.float32 if emit_keys else jnp.int32),
        pltpu.VMEM((_CHUNK,), jnp.float32 if emit_keys else jnp.int32),
        pltpu.VMEM_SHARED((_NS, nbins), jnp.int32),
        pltpu.SemaphoreType.DMA,
        pltpu.SemaphoreType.DMA,
    ]
    if emit_keys:
        scratch += [
            pltpu.VMEM((_CHUNK,), jnp.int32),
            pltpu.VMEM((_CHUNK,), jnp.int32),
            pltpu.SemaphoreType.DMA,
            pltpu.SemaphoreType.DMA,
        ]
        out_type = (
            jax.ShapeDtypeStruct((_NC, nbins), jnp.int32),
            jax.ShapeDtypeStruct((_N,), jnp.int32),
        )
    else:
        out_type = jax.ShapeDtypeStruct((_NC, nbins), jnp.int32)

    @functools.partial(
        pl.kernel,
        out_type=out_type,
        mesh=mesh,
        scratch_types=scratch,
        compiler_params=pltpu.CompilerParams(needs_layout_passes=False),
    )
    def sc_pass(*refs):
        keys_hbm = refs[0]
        prior_hbm = refs[1 : 1 + nprior]
        out_hbm = refs[1 + nprior]
        nout = 2 if emit_keys else 1
        keys_out = refs[2 + nprior] if emit_keys else None
        base_i = 1 + nprior + nout
        prior_v = refs[base_i : base_i + nprior]
        hist, tmp, buf0, buf1, shared = refs[base_i + nprior : base_i + nprior + 5]
        rest = refs[base_i + nprior + 5 :]
        sem0, sem1 = rest[0], rest[1]
        if emit_keys:
            kbufs, ksems = (rest[2], rest[3]), (rest[4], rest[5])
        bufs, sems = (buf0, buf1), (sem0, sem1)

        c = lax.axis_index("c")
        s = lax.axis_index("s")
        zeros = jnp.zeros((_L,), jnp.int32)
        wid = c * _NS + s
        base = wid * _PER_W
        cpr = 32768 // _CHUNK
        rpw = 128 // _NW

        def mk(j, buf, sem):
            if emit_keys:
                row = wid * rpw + j // cpr
                col = (j % cpr) * _CHUNK
                src = keys_hbm.at[row].at[pl.ds(col, _CHUNK)]
            else:
                src = keys_hbm.at[pl.ds(base + j * _CHUNK, _CHUNK)]
            return pltpu.make_async_copy(src, buf, sem)

        bufs, sems = (buf0, buf1), (sem0, sem1)
        mk(0, bufs[0], sems[0]).start()
        mk(1, bufs[1], sems[1]).start()

        def zbody(i, _):
            hist[pl.ds(i * _L, _L)] = zeros
            return 0

        lax.fori_loop(0, rows, zbody, 0)

        k = jnp.int32(_K)
        pfx = jnp.int32(0)
        for p in range(nprior):
            prows = (1 << prior_bits[p]) // _L
            pltpu.sync_copy(prior_hbm[p], prior_v[p])
            pv = prior_v[p]

            def rd(idx, pv=pv):
                return pv[0, pl.ds(idx * _L, _L)] + pv[1, pl.ds(idx * _L, _L)]

            b, k = _sc_pick(rd, prows, k)
            pfx = (pfx << prior_bits[p]) | b

        if nprior:
            ext = 32 - pwidth
            pfx_se = ((pfx ^ (1 << (pwidth - 1))) << ext) >> ext

        ones = zeros + 1
        kcps = [None, None]
        for j in range(_NCHUNK):
            cur = j % 2
            mk(j, bufs[cur], sems[cur]).wait()
            buf = bufs[cur]

            if emit_keys:
                if kcps[cur] is not None:
                    kcps[cur].wait()
                kbuf = kbufs[cur]

                @plsc.parallel_loop(0, _CHUNK, _L, unroll=4)
                def vbody(v, buf=buf, kbuf=kbuf):
                    b = plsc.bitcast(buf[pl.ds(v, _L)], jnp.int32)
                    skey = b ^ ((b >> 31) & jnp.int32(0x7FFFFFFF))
                    kbuf[pl.ds(v, _L)] = skey
                    bin_i = ((skey >> shift) & (nbins - 1)) ^ binflip
                    plsc.addupdate_scatter(hist, [bin_i], ones)

                kcps[cur] = pltpu.make_async_copy(
                    kbuf, keys_out.at[pl.ds(base + j * _CHUNK, _CHUNK)], ksems[cur]
                )
                kcps[cur].start()
            else:

                @plsc.parallel_loop(0, _CHUNK, _L, unroll=4)
                def vbody(v, buf=buf):
                    skey = buf[pl.ds(v, _L)]
                    bin_i = ((skey >> shift) & (nbins - 1)) ^ binflip
                    if nprior:
                        keep = (skey >> pshift) == pfx_se
                        plsc.addupdate_scatter(hist, [bin_i], ones, mask=keep)
                    else:
                        plsc.addupdate_scatter(hist, [bin_i], ones)

            if j + 2 < _NCHUNK:
                mk(j + 2, bufs[cur], sems[cur]).start()

        if emit_keys:
            for kcp in kcps:
                if kcp is not None:
                    kcp.wait()

        pltpu.sync_copy(hist, shared.at[s])
        plsc.subcore_barrier()
        for step in (8, 4, 2, 1):

            @pl.when(s < step)
            def _(step=step):
                pltpu.sync_copy(shared.at[s + step], tmp)

                @plsc.parallel_loop(0, nbins, _L, unroll=4)
                def abody(i):
                    hist[pl.ds(i, _L)] = hist[pl.ds(i, _L)] + tmp[pl.ds(i, _L)]

                pltpu.sync_copy(hist, shared.at[s])

            plsc.subcore_barrier()

        @pl.when(s == 0)
        def _():
            pltpu.sync_copy(hist, out_hbm.at[c])

    return sc_pass


_pass1 = _make_sc_pass([], 22, _B1, emit_keys=True)
_pass2 = _make_sc_pass([_B1], 11, _B2)
_pass3 = _make_sc_pass([_B1, _B2], 0, _B3)


def _tc_pick(h, k):
    nbins = h.shape[1]
    tot = jnp.sum(h, axis=0, keepdims=True)
    iot = lax.broadcasted_iota(jnp.int32, tot.shape, 1)
    zero = jnp.zeros_like(tot)

    def suffix(b):
        return jnp.sum(jnp.where(iot >= b, tot, zero))

    b = jnp.int32(0)
    bit = nbins >> 1
    while bit:
        cand = b + bit
        b = jnp.where(suffix(cand) >= k, cand, b)
        bit >>= 1
    return b, k - suffix(b + 1)


def _mask_kernel(x_ref, h1_ref, h2_ref, h3_ref, o_ref):
    b1, k1 = _tc_pick(h1_ref[...], jnp.int32(_K))
    b2, k2 = _tc_pick(h2_ref[...], k1)
    b3, _ = _tc_pick(h3_ref[...], k2)
    t = (
        (b1.astype(jnp.uint32) << (_B2 + _B3))
        | (b2.astype(jnp.uint32) << _B3)
        | b3.astype(jnp.uint32)
    )
    bits = jnp.where(t >= jnp.uint32(0x80000000), t ^ jnp.uint32(0x80000000), ~t)
    thresh = lax.bitcast_convert_type(bits, jnp.float32)
    o_ref[...] = (x_ref[...] >= thresh).astype(jnp.float32)


def kernel(x):
    h1, keys = _pass1(x)
    h2 = _pass2(keys, h1)
    h3 = _pass3(keys, h1, h2)
    return pl.pallas_call(
        _mask_kernel,
        out_shape=jax.ShapeDtypeStruct(x.shape, jnp.float32),
    )(x, h1, h2, h3)


# speedup vs baseline: 1.0448x; 1.0090x over previous
import functools

import jax
import jax.numpy as jnp
from jax import lax
from jax.experimental import pallas as pl
from jax.experimental.pallas import tpu as pltpu
from jax.experimental.pallas import tpu_sc as plsc

_K = 10000
_NC, _NS, _L = 2, 16, 16
_NW = _NC * _NS
_N = 128 * 32768
_PER_W = _N // _NW


_B1, _B2, _B3 = 10, 11, 11


def _iota16():
    return lax.broadcasted_iota(jnp.int32, (_L,), 0)


def _sc_pick(read_vec, nvec, k):
    def body(i, st):
        found, bstar, carry, snext = st
        idx = nvec - 1 - i
        v = read_vec(idx)
        svec = lax.rev(plsc.cumsum(lax.rev(v, (0,))), (0,)) + carry
        mask = svec >= k
        m = jnp.max(jnp.where(mask, _iota16(), -1))
        sn_v = jnp.max(jnp.where(mask, 0, svec))
        sn_v = jnp.where(m == _L - 1, carry, sn_v)
        hit = jnp.logical_and(m >= 0, found == 0)
        bstar = jnp.where(hit, idx * _L + m, bstar)
        snext = jnp.where(hit, sn_v, snext)
        found = jnp.where(m >= 0, 1, found)
        carry = svec[0]
        return found, bstar, carry, snext

    _, bstar, _, snext = lax.fori_loop(
        0, nvec, body, (jnp.int32(0), jnp.int32(0), jnp.int32(0), jnp.int32(0))
    )
    return bstar, k - snext


def _make_sc_pass(prior_bits, shift, bits, emit_keys=False):
    nbins = 1 << bits
    rows = nbins // _L
    nprior = len(prior_bits)
    _CHUNK = 16384
    _NCHUNK = _PER_W // _CHUNK
    pshift = shift + bits
    pwidth = 32 - pshift
    binflip = (nbins >> 1) if shift + bits == 32 else 0
    mesh = plsc.VectorSubcoreMesh(core_axis_name="c", subcore_axis_name="s")

    scratch = [
        pltpu.VMEM((_NC, 1 << pb), jnp.int32) for pb in prior_bits
    ]
    scratch += [
        pltpu.VMEM((nbins,), jnp.int32),
        pltpu.VMEM((nbins,), jnp.int32),
        pltpu.VMEM((_CHUNK,), jnp.float32 if emit_keys else jnp.int32),
        pltpu.VMEM((_CHUNK,), jnp.float32 if emit_keys else jnp.int32),
        pltpu.VMEM_SHARED((_NS, nbins), jnp.int32),
        pltpu.SemaphoreType.DMA,
        pltpu.SemaphoreType.DMA,
    ]
    if emit_keys:
        scratch += [
            pltpu.VMEM((_CHUNK,), jnp.int32),
            pltpu.VMEM((_CHUNK,), jnp.int32),
            pltpu.SemaphoreType.DMA,
            pltpu.SemaphoreType.DMA,
        ]
        out_type = (
            jax.ShapeDtypeStruct((_NC, nbins), jnp.int32),
            jax.ShapeDtypeStruct((_N,), jnp.int32),
        )
    else:
        out_type = jax.ShapeDtypeStruct((_NC, nbins), jnp.int32)

    @functools.partial(
        pl.kernel,
        out_type=out_type,
        mesh=mesh,
        scratch_types=scratch,
        compiler_params=pltpu.CompilerParams(needs_layout_passes=False),
    )
    def sc_pass(*refs):
        keys_hbm = refs[0]
        prior_hbm = refs[1 : 1 + nprior]
        out_hbm = refs[1 + nprior]
        nout = 2 if emit_keys else 1
        keys_out = refs[2 + nprior] if emit_keys else None
        base_i = 1 + nprior + nout
        prior_v = refs[base_i : base_i + nprior]
        hist, tmp, buf0, buf1, shared = refs[base_i + nprior : base_i + nprior + 5]
        rest = refs[base_i + nprior + 5 :]
        sem0, sem1 = rest[0], rest[1]
        if emit_keys:
            kbufs, ksems = (rest[2], rest[3]), (rest[4], rest[5])
        bufs, sems = (buf0, buf1), (sem0, sem1)

        c = lax.axis_index("c")
        s = lax.axis_index("s")
        zeros = jnp.zeros((_L,), jnp.int32)
        wid = c * _NS + s
        base = wid * _PER_W
        cpr = 32768 // _CHUNK
        rpw = 128 // _NW

        def mk(j, buf, sem):
            if emit_keys:
                row = wid * rpw + j // cpr
                col = (j % cpr) * _CHUNK
                src = keys_hbm.at[row].at[pl.ds(col, _CHUNK)]
            else:
                src = keys_hbm.at[pl.ds(base + j * _CHUNK, _CHUNK)]
            return pltpu.make_async_copy(src, buf, sem)

        bufs, sems = (buf0, buf1), (sem0, sem1)
        mk(0, bufs[0], sems[0]).start()
        mk(1, bufs[1], sems[1]).start()

        def zbody(i, _):
            hist[pl.ds(i * _L, _L)] = zeros
            return 0

        lax.fori_loop(0, rows, zbody, 0)

        k = jnp.int32(_K)
        pfx = jnp.int32(0)
        for p in range(nprior):
            prows = (1 << prior_bits[p]) // _L
            pltpu.sync_copy(prior_hbm[p], prior_v[p])
            pv = prior_v[p]

            def rd(idx, pv=pv):
                return pv[0, pl.ds(idx * _L, _L)] + pv[1, pl.ds(idx * _L, _L)]

            b, k = _sc_pick(rd, prows, k)
            pfx = (pfx << prior_bits[p]) | b

        if nprior:
            ext = 32 - pwidth
            pfx_se = ((pfx ^ (1 << (pwidth - 1))) << ext) >> ext

        ones = zeros + 1
        kcps = [None, None]
        for j in range(_NCHUNK):
            cur = j % 2
            mk(j, bufs[cur], sems[cur]).wait()
            buf = bufs[cur]

            if emit_keys:
                if kcps[cur] is not None:
                    kcps[cur].wait()
                kbuf = kbufs[cur]

                @plsc.parallel_loop(0, _CHUNK, _L, unroll=4)
                def vbody(v, buf=buf, kbuf=kbuf):
                    b = plsc.bitcast(buf[pl.ds(v, _L)], jnp.int32)
                    skey = b ^ ((b >> 31) & jnp.int32(0x7FFFFFFF))
                    kbuf[pl.ds(v, _L)] = skey
                    bin_i = ((skey >> shift) & (nbins - 1)) ^ binflip
                    plsc.addupdate_scatter(hist, [bin_i], ones)

                kcps[cur] = pltpu.make_async_copy(
                    kbuf, keys_out.at[pl.ds(base + j * _CHUNK, _CHUNK)], ksems[cur]
                )
                kcps[cur].start()
            else:

                @plsc.parallel_loop(0, _CHUNK, _L, unroll=4)
                def vbody(v, buf=buf):
                    skey = buf[pl.ds(v, _L)]
                    bin_i = ((skey >> shift) & (nbins - 1)) ^ binflip
                    if nprior:
                        keep = (skey >> pshift) == pfx_se
                        plsc.addupdate_scatter(hist, [bin_i], ones, mask=keep)
                    else:
                        plsc.addupdate_scatter(hist, [bin_i], ones)

            if j + 2 < _NCHUNK:
                mk(j + 2, bufs[cur], sems[cur]).start()

        if emit_keys:
            for kcp in kcps:
                if kcp is not None:
                    kcp.wait()

        pltpu.sync_copy(hist, shared.at[s])
        plsc.subcore_barrier()
        for step in (8, 4, 2, 1):

            @pl.when(s < step)
            def _(step=step):
                pltpu.sync_copy(shared.at[s + step], tmp)

                @plsc.parallel_loop(0, nbins, _L, unroll=4)
                def abody(i):
                    hist[pl.ds(i, _L)] = hist[pl.ds(i, _L)] + tmp[pl.ds(i, _L)]

                pltpu.sync_copy(hist, shared.at[s])

            plsc.subcore_barrier()

        @pl.when(s == 0)
        def _():
            pltpu.sync_copy(hist, out_hbm.at[c])

    return sc_pass


_pass1 = _make_sc_pass([], 22, _B1, emit_keys=True)
_pass2 = _make_sc_pass([_B1], 11, _B2)
_pass3 = _make_sc_pass([_B1, _B2], 0, _B3)


def _tc_pick(h, k):
    nbins = h.shape[1]
    tot = jnp.sum(h, axis=0, keepdims=True)
    iot = lax.broadcasted_iota(jnp.int32, tot.shape, 1)
    zero = jnp.zeros_like(tot)

    def suffix(b):
        return jnp.sum(jnp.where(iot >= b, tot, zero))

    b = jnp.int32(0)
    bit = nbins >> 1
    while bit:
        cand = b + bit
        b = jnp.where(suffix(cand) >= k, cand, b)
        bit >>= 1
    return b, k - suffix(b + 1)


def _mask_kernel(x_ref, h1_ref, h2_ref, h3_ref, o_ref):
    b1, k1 = _tc_pick(h1_ref[...], jnp.int32(_K))
    b2, k2 = _tc_pick(h2_ref[...], k1)
    b3, _ = _tc_pick(h3_ref[...], k2)
    t = (
        (b1.astype(jnp.uint32) << (_B2 + _B3))
        | (b2.astype(jnp.uint32) << _B3)
        | b3.astype(jnp.uint32)
    )
    bits = jnp.where(t >= jnp.uint32(0x80000000), t ^ jnp.uint32(0x80000000), ~t)
    thresh = lax.bitcast_convert_type(bits, jnp.float32)
    o_ref[...] = (x_ref[...] >= thresh).astype(jnp.float32)


def kernel(x):
    h1, keys = _pass1(x)
    h2 = _pass2(keys, h1)
    h3 = _pass3(keys, h1, h2)
    return pl.pallas_call(
        _mask_kernel,
        out_shape=jax.ShapeDtypeStruct(x.shape, jnp.float32),
    )(x, h1, h2, h3)
